# baseline (speedup 1.0000x reference)
import jax
import jax.numpy as jnp
from jax.experimental import pallas as pl
from jax.experimental.pallas import tpu as pltpu

B, S, H = 16, 4096, 1024
D = 64
W = 2 * D
WP = W + 8
f32 = jnp.float32
bf16 = jnp.bfloat16


def _nmt_kernel(hbm_ref, yt_ref, len_ref, wtan_hbm, wpt_ref, wct_hbm,
                out_ref, gath, startv, starts, wtan_vmem, wct_vmem,
                sem, wsem, wtsems, gsems):
    KH = H // 2
    wt_cps = []
    for i in range(2):
        c = pltpu.make_async_copy(
            wtan_hbm.at[pl.ds(i * KH, KH), :],
            wtan_vmem.at[pl.ds(i * KH, KH), :], wtsems.at[i])
        c.start()
        wt_cps.append(c)
    wcp = pltpu.make_async_copy(wct_hbm, wct_vmem, wsem)
    wcp.start()

    yt16 = yt_ref[:].astype(bf16)
    wt_cps[0].wait()
    z1 = jnp.dot(yt16[:, :KH], wtan_vmem[0:KH, :].astype(bf16),
                 preferred_element_type=f32)
    wt_cps[1].wait()
    z1 = z1 + jnp.dot(yt16[:, KH:], wtan_vmem[KH:, :].astype(bf16),
                      preferred_element_type=f32)
    t16 = jnp.tanh(z1).astype(bf16)
    logit = jnp.dot(t16, wpt_ref[:].astype(bf16), preferred_element_type=f32)
    lens_i = len_ref[:]
    pt = jax.nn.sigmoid(logit) * lens_i.astype(f32)
    pti = jnp.floor(pt).astype(jnp.int32)
    left = jnp.maximum(0, pti - D)
    right = jnp.minimum(lens_i, pti + D)
    start = jnp.minimum((left // 8) * 8, S - WP)

    startv[:] = start

    RS = 64
    copies = []
    for b in range(B):
        s = pl.multiple_of(startv[b, 0], 8)
        c0 = pltpu.make_async_copy(
            hbm_ref.at[b, pl.ds(s, RS), :],
            gath.at[b, pl.ds(0, RS), :], gsems.at[b, 0])
        c0.start()
        c1 = pltpu.make_async_copy(
            hbm_ref.at[b, pl.ds(s + RS, WP - RS), :],
            gath.at[b, pl.ds(RS, WP - RS), :], gsems.at[b, 1])
        c1.start()
        copies.append((c0, c1))

    G = 4
    score_parts, g16_parts = [], []
    for gi in range(0, B, G):
        for b in range(gi, gi + G):
            copies[b][0].wait()
            copies[b][1].wait()
        gp16 = gath[gi:gi + G].astype(bf16)
        g16_parts.append(gp16)
        score_parts.append(jax.lax.dot_general(
            gp16, yt16[gi:gi + G], (((2,), (1,)), ((0,), (0,))),
            preferred_element_type=f32))
    scores = jnp.concatenate(score_parts, axis=0)
    g16 = jnp.concatenate(g16_parts, axis=0)

    jpos = start + jax.lax.broadcasted_iota(jnp.int32, (B, WP), 1)
    valid = (jpos >= left) & (jpos < right)
    scores = jnp.where(valid, scores, -1e30)
    m = jnp.max(scores, axis=1, keepdims=True)
    e = jnp.exp(scores - m)
    align = e / jnp.sum(e, axis=1, keepdims=True)
    d = jpos.astype(f32) - pt
    ex_p = jnp.exp(-(d * d) / (D * D / 2.0))
    at16 = (align * ex_p * valid.astype(f32)).astype(bf16)

    ct = jax.lax.dot_general(
        at16, g16, (((1,), (1,)), ((0,), (0,))),
        preferred_element_type=f32).astype(bf16)
    wcp.wait()
    out_ref[:] = jnp.dot(ct, wct_vmem[:].astype(bf16),
                         preferred_element_type=f32)


def kernel(encode_h, yt, encode_len, W_tan, w_pt, W_ct2ht):
    return pl.pallas_call(
        _nmt_kernel,
        in_specs=[
            pl.BlockSpec(memory_space=pl.ANY),
            pl.BlockSpec(memory_space=pltpu.MemorySpace.VMEM),
            pl.BlockSpec(memory_space=pltpu.MemorySpace.VMEM),
            pl.BlockSpec(memory_space=pl.ANY),
            pl.BlockSpec(memory_space=pltpu.MemorySpace.VMEM),
            pl.BlockSpec(memory_space=pl.ANY),
        ],
        out_shape=jax.ShapeDtypeStruct((B, H), f32),
        scratch_shapes=[
            pltpu.VMEM((B, WP, H), f32),
            pltpu.VMEM((B, 1), jnp.int32),
            pltpu.SMEM((B, 1), jnp.int32),
            pltpu.VMEM((H, H), f32),
            pltpu.VMEM((H, H), f32),
            pltpu.SemaphoreType.DMA,
            pltpu.SemaphoreType.DMA,
            pltpu.SemaphoreType.DMA((2,)),
            pltpu.SemaphoreType.DMA((B, 2)),
        ],
    )(encode_h, yt, encode_len.reshape(B, 1), W_tan, w_pt, W_ct2ht)


# speedup vs baseline: 1.0261x; 1.0261x over previous
import jax
import jax.numpy as jnp
from jax.experimental import pallas as pl
from jax.experimental.pallas import tpu as pltpu

B, S, H = 16, 4096, 1024
D = 64
W = 2 * D
WP = W + 8
f32 = jnp.float32
bf16 = jnp.bfloat16


def _nmt_kernel(hbm_ref, yt_ref, len_ref, wtan_ref, wpt_ref, wct_hbm,
                out_ref, gath, startv, wct_vmem, wsem, gsems):
    wcp = pltpu.make_async_copy(wct_hbm, wct_vmem, wsem)
    wcp.start()

    yt16 = yt_ref[:].astype(bf16)
    z1 = jnp.dot(yt16, wtan_ref[:].astype(bf16), preferred_element_type=f32)
    t16 = jnp.tanh(z1).astype(bf16)
    logit = jnp.dot(t16, wpt_ref[:].astype(bf16), preferred_element_type=f32)
    lens_i = len_ref[:]
    pt = jax.nn.sigmoid(logit) * lens_i.astype(f32)
    pti = jnp.floor(pt).astype(jnp.int32)
    left = jnp.maximum(0, pti - D)
    right = jnp.minimum(lens_i, pti + D)
    start = jnp.minimum((left // 8) * 8, S - WP)

    startv[:] = start

    copies = []
    for b in range(B):
        s = pl.multiple_of(startv[b, 0], 8)
        c = pltpu.make_async_copy(
            hbm_ref.at[b, pl.ds(s, WP), :], gath.at[b], gsems.at[b])
        c.start()
        copies.append(c)

    G = 4
    score_parts, g16_parts = [], []
    for gi in range(0, B, G):
        for b in range(gi, gi + G):
            copies[b].wait()
        gp16 = gath[gi:gi + G].astype(bf16)
        g16_parts.append(gp16)
        score_parts.append(jax.lax.dot_general(
            gp16, yt16[gi:gi + G], (((2,), (1,)), ((0,), (0,))),
            preferred_element_type=f32))
    scores = jnp.concatenate(score_parts, axis=0)
    g16 = jnp.concatenate(g16_parts, axis=0)

    jpos = start + jax.lax.broadcasted_iota(jnp.int32, (B, WP), 1)
    valid = (jpos >= left) & (jpos < right)
    scores = jnp.where(valid, scores, -1e30)
    m = jnp.max(scores, axis=1, keepdims=True)
    e = jnp.exp(scores - m)
    align = e / jnp.sum(e, axis=1, keepdims=True)
    d = jpos.astype(f32) - pt
    ex_p = jnp.exp(-(d * d) / (D * D / 2.0))
    at16 = (align * ex_p * valid.astype(f32)).astype(bf16)

    ct = jax.lax.dot_general(
        at16, g16, (((1,), (1,)), ((0,), (0,))),
        preferred_element_type=f32).astype(bf16)
    wcp.wait()
    out_ref[:] = jnp.dot(ct, wct_vmem[:].astype(bf16),
                         preferred_element_type=f32)


def kernel(encode_h, yt, encode_len, W_tan, w_pt, W_ct2ht):
    return pl.pallas_call(
        _nmt_kernel,
        in_specs=[
            pl.BlockSpec(memory_space=pl.ANY),
            pl.BlockSpec(memory_space=pltpu.MemorySpace.VMEM),
            pl.BlockSpec(memory_space=pltpu.MemorySpace.VMEM),
            pl.BlockSpec(memory_space=pltpu.MemorySpace.VMEM),
            pl.BlockSpec(memory_space=pltpu.MemorySpace.VMEM),
            pl.BlockSpec(memory_space=pl.ANY),
        ],
        out_shape=jax.ShapeDtypeStruct((B, H), f32),
        scratch_shapes=[
            pltpu.VMEM((B, WP, H), f32),
            pltpu.VMEM((B, 1), jnp.int32),
            pltpu.VMEM((H, H), f32),
            pltpu.SemaphoreType.DMA,
            pltpu.SemaphoreType.DMA((B,)),
        ],
    )(encode_h, yt, encode_len.reshape(B, 1), W_tan, w_pt, W_ct2ht)
